# baseline (speedup 1.0000x reference)
import functools

import jax
import jax.numpy as jnp
from jax.experimental import pallas as pl
from jax.experimental.pallas import tpu as pltpu


def _body(a_ref, x_ref, wt_ref, wu_ref, bu_ref, out_ref, tp_ref, xu_ref,
          *, bm, d, units):
    i = pl.program_id(0)

    @pl.when(i == 0)
    def _():
        n = x_ref.shape[0]
        wu = wu_ref[:]
        t = jnp.dot(x_ref[:], wt_ref[:], preferred_element_type=jnp.float32)
        tp = jnp.dot(t, wu[d:, :], preferred_element_type=jnp.float32)
        ones_col = (jax.lax.broadcasted_iota(jnp.int32, (n, 128), 1) == 0)
        tp_ref[:] = jnp.concatenate([tp, ones_col.astype(jnp.float32)], axis=1)
        xu_ref[:] = (jnp.dot(x_ref[:], wu[:d, :],
                             preferred_element_type=jnp.float32) + bu_ref[:])

    res = jnp.dot(a_ref[:], tp_ref[:], preferred_element_type=jnp.float32)
    agg = res[:, :units]
    deg = res[:, units:units + 1]
    dinv = jnp.where(deg == 0.0, 0.0, 1.0 / deg)
    xu_band = xu_ref[pl.ds(i * bm, bm), :]
    out_ref[:] = jnp.maximum(xu_band + agg * dinv, 0.0)


@jax.jit
def kernel(node_features, adjacency, W_t, W_u, b_u):
    n, d = node_features.shape
    units = W_t.shape[1]
    bm = 400
    nm = n // bm

    return pl.pallas_call(
        functools.partial(_body, bm=bm, d=d, units=units),
        grid=(nm,),
        in_specs=[
            pl.BlockSpec((bm, n), lambda i: (i, 0)),
            pl.BlockSpec((n, d), lambda i: (0, 0)),
            pl.BlockSpec((d, units), lambda i: (0, 0)),
            pl.BlockSpec((d + units, units), lambda i: (0, 0)),
            pl.BlockSpec((1, units), lambda i: (0, 0)),
        ],
        out_specs=pl.BlockSpec((bm, units), lambda i: (i, 0)),
        out_shape=jax.ShapeDtypeStruct((n, units), jnp.float32),
        scratch_shapes=[
            pltpu.VMEM((n, units + 128), jnp.float32),
            pltpu.VMEM((n, units), jnp.float32),
        ],
        compiler_params=pltpu.CompilerParams(
            dimension_semantics=("arbitrary",),
        ),
    )(adjacency, node_features, W_t, W_u, b_u.reshape(1, units))


# speedup vs baseline: 1.0060x; 1.0060x over previous
import functools

import jax
import jax.numpy as jnp
from jax.experimental import pallas as pl
from jax.experimental.pallas import tpu as pltpu


def _body(a_ref, x_ref, wt_ref, wu_ref, bu_ref, out_ref, t_ref, *, bm, d, units):
    i = pl.program_id(0)

    @pl.when(i == 0)
    def _():
        n = x_ref.shape[0]
        t = jnp.dot(x_ref[:], wt_ref[:], preferred_element_type=jnp.float32)
        ones_col = (jax.lax.broadcasted_iota(jnp.int32, (n, 128), 1) == 0)
        t_ref[:] = jnp.concatenate([t, ones_col.astype(jnp.float32)], axis=1)

    res = jnp.dot(a_ref[:], t_ref[:], preferred_element_type=jnp.float32)
    acc = res[:, :units]
    deg = res[:, units:units + 1]
    dinv = jnp.where(deg == 0.0, 0.0, 1.0 / deg)
    nb = acc * dinv
    x_band = x_ref[pl.ds(i * bm, bm), :]
    wu = wu_ref[:]
    out = (jnp.dot(x_band, wu[:d, :], preferred_element_type=jnp.float32)
           + jnp.dot(nb, wu[d:, :], preferred_element_type=jnp.float32)
           + bu_ref[:])
    out_ref[:] = jnp.maximum(out, 0.0)


@jax.jit
def kernel(node_features, adjacency, W_t, W_u, b_u):
    n, d = node_features.shape
    units = W_t.shape[1]
    bm = 400
    nm = n // bm

    return pl.pallas_call(
        functools.partial(_body, bm=bm, d=d, units=units),
        grid=(nm,),
        in_specs=[
            pl.BlockSpec((bm, n), lambda i: (i, 0)),
            pl.BlockSpec((n, d), lambda i: (0, 0)),
            pl.BlockSpec((d, units), lambda i: (0, 0)),
            pl.BlockSpec((d + units, units), lambda i: (0, 0)),
            pl.BlockSpec((1, units), lambda i: (0, 0)),
        ],
        out_specs=pl.BlockSpec((bm, units), lambda i: (i, 0)),
        out_shape=jax.ShapeDtypeStruct((n, units), jnp.float32),
        scratch_shapes=[pltpu.VMEM((n, units + 128), jnp.float32)],
        compiler_params=pltpu.CompilerParams(
            dimension_semantics=("arbitrary",),
        ),
    )(adjacency, node_features, W_t, W_u, b_u.reshape(1, units))
